# Initial kernel scaffold; baseline (speedup 1.0000x reference)
import jax
import jax.numpy as jnp
from jax.experimental import pallas as pl


def kernel(H, edge_index, edge_type, node_type, node_score, Wnt, bnt, Wsc, bsc, We1, be1, We2, be2, Wk, bk, Wm, bm, Wq, bq, W1, b1, W2, b2, VhW, Vhb, VxW, Vxb):
    raise NotImplementedError("write your pallas kernel here")


# speedup vs baseline: 1.6278x; 1.6278x over previous
import functools
import math

import numpy as np
import jax
import jax.numpy as jnp
from jax import lax
from jax.experimental import pallas as pl
from jax.experimental.pallas import tpu as pltpu
from jax.experimental.pallas import tpu_sc as plsc

HID = 128
HEADS = 4
DPH = 32
NTY = 4
NETY = 38
NCLS = (NETY + 1) * NTY * NTY
NNODE = 10000
NEDGE = 320000
ETOT = NEDGE + NNODE
NC = 2
NS = 16
LANES = 16
NW = NC * NS
CH = 128
NCHUNK = -(-ETOT // (NW * CH))
PERW = NCHUNK * CH
EPAD = PERW * NW
CHC = 64
NCHUNKC = PERW // CHC
EPS = 1e-16
INV_SQRT_DPH = 1.0 / math.sqrt(DPH)


def _gelu(x):
    c = math.sqrt(2.0 / math.pi)
    return 0.5 * x * (1.0 + jnp.tanh(c * (x + 0.044715 * x * x * x)))


def _prep_body(nt_ref, ns_ref, js_ref, eein_ref, Wnt_ref, bnt_ref, Wsc_ref,
               bsc_ref, We1_ref, be1_ref, We2_ref, be2_ref, nfe_ref, emb_ref):
    nt = nt_ref[...]
    T = (nt == lax.broadcasted_iota(jnp.int32, (NNODE, NTY), 1)).astype(jnp.float32)
    nte = _gelu(jnp.dot(T, Wnt_ref[...], preferred_element_type=jnp.float32) + bnt_ref[...])
    Bmat = jnp.sin(ns_ref[...] * js_ref[...])
    nse = _gelu(jnp.dot(Bmat, Wsc_ref[...], preferred_element_type=jnp.float32) + bsc_ref[...])
    nfe_ref[...] = jnp.concatenate([nte, nse], axis=1)
    h1 = jnp.maximum(jnp.dot(eein_ref[...], We1_ref[...], preferred_element_type=jnp.float32) + be1_ref[...], 0.0)
    emb_ref[...] = jnp.dot(h1, We2_ref[...], preferred_element_type=jnp.float32) + be2_ref[...]


_prep = pl.pallas_call(
    _prep_body,
    out_shape=(jax.ShapeDtypeStruct((NNODE, HID), jnp.float32),
               jax.ShapeDtypeStruct((NCLS, HID), jnp.float32)),
)


def _tables_body(x_ref, nfe_ref, Wq_ref, bq_ref, Wkx_ref, bk_ref, Wke_ref,
                 Wmx_ref, bm_ref, Wme_ref, emb_ref,
                 qn_ref, kn_ref, mn_ref, ke_ref, me_ref):
    x2 = jnp.concatenate([x_ref[...], nfe_ref[...]], axis=1)
    qn_ref[...] = (jnp.dot(x2, Wq_ref[...], preferred_element_type=jnp.float32) + bq_ref[...]) * INV_SQRT_DPH
    kn_ref[...] = jnp.dot(x2, Wkx_ref[...], preferred_element_type=jnp.float32) + bk_ref[...]
    mn_ref[...] = jnp.dot(x2, Wmx_ref[...], preferred_element_type=jnp.float32) + bm_ref[...]
    ke_ref[...] = jnp.dot(emb_ref[...], Wke_ref[...], preferred_element_type=jnp.float32)
    me_ref[...] = jnp.dot(emb_ref[...], Wme_ref[...], preferred_element_type=jnp.float32)


_tables = pl.pallas_call(
    _tables_body,
    out_shape=(jax.ShapeDtypeStruct((NNODE, HID), jnp.float32),
               jax.ShapeDtypeStruct((NNODE, HID), jnp.float32),
               jax.ShapeDtypeStruct((NNODE, HID), jnp.float32),
               jax.ShapeDtypeStruct((NCLS, HID), jnp.float32),
               jax.ShapeDtypeStruct((NCLS, HID), jnp.float32)),
)


def _post_body(a_ref, W1_ref, b1_ref, W2_ref, b2_ref, x_ref):
    a = a_ref[0] + a_ref[1]
    h = jnp.maximum(jnp.dot(a, W1_ref[...], preferred_element_type=jnp.float32) + b1_ref[...], 0.0)
    x_ref[...] = _gelu(jnp.dot(h, W2_ref[...], preferred_element_type=jnp.float32) + b2_ref[...])


_post = pl.pallas_call(
    _post_body,
    out_shape=jax.ShapeDtypeStruct((NNODE, HID), jnp.float32),
)


def _final_body(h_ref, x_ref, VhW_ref, Vhb_ref, VxW_ref, Vxb_ref, o_ref):
    o_ref[...] = _gelu(
        jnp.dot(h_ref[...], VhW_ref[...], preferred_element_type=jnp.float32) + Vhb_ref[...]
        + jnp.dot(x_ref[...], VxW_ref[...], preferred_element_type=jnp.float32) + Vxb_ref[...])


_final = pl.pallas_call(
    _final_body,
    out_shape=jax.ShapeDtypeStruct((NNODE, HID), jnp.float32),
)


_sc_mesh = plsc.VectorSubcoreMesh(core_axis_name="c", subcore_axis_name="s",
                                  num_cores=NC, num_subcores=NS)


def _worker_base():
    cid = lax.axis_index("c")
    sid = lax.axis_index("s")
    wid = sid * NC + cid
    return cid, sid, wid * PERW


def _iota16():
    return lax.iota(jnp.int32, LANES)


def _cls_body(src_hbm, dst_hbm, et_hbm, ntf_hbm,
              cls_hbm, cnt_hbm,
              src_v, dst_v, et_v, cls_v, nt_v, cnt_v):
    cid, sid, base0 = _worker_base()
    wid = sid * NC + cid
    pltpu.sync_copy(ntf_hbm, nt_v)

    def zinit(i, c):
        cnt_v[pl.ds(i * LANES, LANES)] = jnp.zeros((LANES,), jnp.float32)
        return c

    lax.fori_loop(0, NNODE // LANES, zinit, 0)

    def chunk(i, carry):
        base = base0 + i * CH
        pltpu.sync_copy(src_hbm.at[pl.ds(base, CH)], src_v)
        pltpu.sync_copy(dst_hbm.at[pl.ds(base, CH)], dst_v)
        pltpu.sync_copy(et_hbm.at[pl.ds(base, CH)], et_v)

        def grp(g, c2):
            sl = pl.ds(g * LANES, LANES)
            erows = g * LANES + _iota16()
            s16 = src_v[sl]
            d16 = dst_v[sl]
            nts = plsc.load_gather(nt_v, [s16])
            ntd = plsc.load_gather(nt_v, [d16])
            cls_v[sl] = et_v[sl] * (NTY * NTY) + nts * NTY + ntd
            valid = (base + erows) < ETOT
            ones = jnp.where(valid, 1.0, 0.0)
            plsc.addupdate_scatter(cnt_v, [s16], ones)
            return c2

        lax.fori_loop(0, CH // LANES, grp, 0)
        pltpu.sync_copy(cls_v, cls_hbm.at[pl.ds(base, CH)])
        return carry

    lax.fori_loop(0, NCHUNK, chunk, 0)
    pltpu.sync_copy(cnt_v, cnt_hbm.at[pl.ds(wid * NNODE, NNODE)])


_cls_call = pl.kernel(
    _cls_body,
    out_type=(jax.ShapeDtypeStruct((EPAD,), jnp.int32),
              jax.ShapeDtypeStruct((NW * NNODE,), jnp.float32)),
    mesh=_sc_mesh,
    compiler_params=pltpu.CompilerParams(needs_layout_passes=False),
    scratch_types=[
        pltpu.VMEM((CH,), jnp.int32),
        pltpu.VMEM((CH,), jnp.int32),
        pltpu.VMEM((CH,), jnp.int32),
        pltpu.VMEM((CH,), jnp.int32),
        pltpu.VMEM((NNODE,), jnp.int32),
        pltpu.VMEM((NNODE,), jnp.float32),
    ],
)


def _score_body(src_hbm, dst_hbm, cls_hbm, qn_hbm, kn_hbm, ke_hbm,
                sc_hbm, smax_hbm,
                src_v, dst_v, cls_v, qb, kb, keb, sb, mx_v, sem):
    cid, sid, base0 = _worker_base()

    def chunk(i, mx):
        base = base0 + i * CH
        pltpu.sync_copy(src_hbm.at[pl.ds(base, CH)], src_v)
        pltpu.sync_copy(dst_hbm.at[pl.ds(base, CH)], dst_v)
        pltpu.sync_copy(cls_hbm.at[pl.ds(base, CH)], cls_v)
        d1 = pltpu.async_copy(qn_hbm.at[src_v], qb, sem)
        d2 = pltpu.async_copy(kn_hbm.at[dst_v], kb, sem)
        d3 = pltpu.async_copy(ke_hbm.at[cls_v], keb, sem)
        d1.wait()
        d2.wait()
        d3.wait()

        def grp(g, mx_g):
            erows = g * LANES + _iota16()
            for h in range(HEADS):
                acc = jnp.zeros((LANES,), jnp.float32)
                for dd in range(DPH):
                    col = jnp.full((LANES,), h * DPH + dd, jnp.int32)
                    qv = plsc.load_gather(qb, [erows, col])
                    kv = plsc.load_gather(kb, [erows, col]) + plsc.load_gather(keb, [erows, col])
                    acc = acc + qv * kv
                sb[h, pl.ds(g * LANES, LANES)] = acc
                mx_g = jnp.maximum(mx_g, acc)
            return mx_g

        mx = lax.fori_loop(0, CH // LANES, grp, mx)
        for h in range(HEADS):
            pltpu.sync_copy(sb.at[h], sc_hbm.at[h, pl.ds(base, CH)])
        return mx

    mx = lax.fori_loop(0, NCHUNK, chunk, jnp.full((LANES,), -1e30, jnp.float32))
    mx_v[...] = mx
    wid = sid * NC + cid
    pltpu.sync_copy(mx_v, smax_hbm.at[pl.ds(wid * LANES, LANES)])


_score_call = pl.kernel(
    _score_body,
    out_type=(jax.ShapeDtypeStruct((HEADS, EPAD), jnp.float32),
              jax.ShapeDtypeStruct((NW * LANES,), jnp.float32)),
    mesh=_sc_mesh,
    compiler_params=pltpu.CompilerParams(needs_layout_passes=False),
    scratch_types=[
        pltpu.VMEM((CH,), jnp.int32),
        pltpu.VMEM((CH,), jnp.int32),
        pltpu.VMEM((CH,), jnp.int32),
        pltpu.VMEM((CH, HID), jnp.float32),
        pltpu.VMEM((CH, HID), jnp.float32),
        pltpu.VMEM((CH, HID), jnp.float32),
        pltpu.VMEM((HEADS, CH), jnp.float32),
        pltpu.VMEM((LANES,), jnp.float32),
        pltpu.SemaphoreType.DMA,
    ],
)


def _denom_body(sc_hbm, src_hbm, cvec_hbm,
                den_hbm,
                src_v, sb, cv_v, den_v):
    cid, sid, base0 = _worker_base()
    wid = sid * NC + cid
    pltpu.sync_copy(cvec_hbm, cv_v)

    def zinit(i, c):
        den_v[pl.ds(i * LANES, LANES)] = jnp.zeros((LANES,), jnp.float32)
        return c

    lax.fori_loop(0, NNODE * HEADS // LANES, zinit, 0)

    def chunk(i, carry):
        base = base0 + i * CH
        pltpu.sync_copy(src_hbm.at[pl.ds(base, CH)], src_v)
        for h in range(HEADS):
            pltpu.sync_copy(sc_hbm.at[h, pl.ds(base, CH)], sb.at[h])
        cv = cv_v[...]

        def grp(g, c2):
            erows = g * LANES + _iota16()
            valid = (base + erows) < ETOT
            s16 = src_v[pl.ds(g * LANES, LANES)]
            sidx = s16 * HEADS
            for h in range(HEADS):
                ex = jnp.exp(sb[h, pl.ds(g * LANES, LANES)] - cv)
                ex = jnp.where(valid, ex, 0.0)
                plsc.addupdate_scatter(den_v, [sidx + h], ex)
            return c2

        lax.fori_loop(0, CH // LANES, grp, 0)
        return carry

    lax.fori_loop(0, NCHUNK, chunk, 0)
    pltpu.sync_copy(den_v, den_hbm.at[pl.ds(wid * NNODE * HEADS, NNODE * HEADS)])


_denom_call = pl.kernel(
    _denom_body,
    out_type=jax.ShapeDtypeStruct((NW * NNODE * HEADS,), jnp.float32),
    mesh=_sc_mesh,
    compiler_params=pltpu.CompilerParams(needs_layout_passes=False),
    scratch_types=[
        pltpu.VMEM((CH,), jnp.int32),
        pltpu.VMEM((HEADS, CH), jnp.float32),
        pltpu.VMEM((LANES,), jnp.float32),
        pltpu.VMEM((NNODE * HEADS,), jnp.float32),
    ],
)


def _apply_body(src_hbm, dst_hbm, cls_hbm, sc_hbm, cvec_hbm, stats_hbm,
                mn_hbm, me_hbm, zn128_hbm,
                aggr2_hbm,
                src_v, dst_v, cls_v, srow_v, sb, stb, mb, meb, ob, cv_v, aggr_sp, sem):
    cid, sid, base0 = _worker_base()
    pltpu.sync_copy(cvec_hbm, cv_v)

    @pl.when(sid == 0)
    def _():
        pltpu.sync_copy(zn128_hbm, aggr_sp)

    plsc.subcore_barrier()

    def chunk(i, carry):
        base = base0 + i * CHC
        pltpu.sync_copy(src_hbm.at[pl.ds(base, CHC)], src_v)
        pltpu.sync_copy(dst_hbm.at[pl.ds(base, CHC)], dst_v)
        pltpu.sync_copy(cls_hbm.at[pl.ds(base, CHC)], cls_v)
        for h in range(HEADS):
            pltpu.sync_copy(sc_hbm.at[h, pl.ds(base, CHC)], sb.at[h])

        def rowidx(g, c2):
            sl = pl.ds(g * LANES, LANES)
            srow_v[sl] = src_v[sl] >> 4
            return c2

        lax.fori_loop(0, CHC // LANES, rowidx, 0)
        d0 = pltpu.async_copy(stats_hbm.at[srow_v], stb, sem)
        d1 = pltpu.async_copy(mn_hbm.at[src_v], mb, sem)
        d2 = pltpu.async_copy(me_hbm.at[cls_v], meb, sem)
        d0.wait()
        d1.wait()
        d2.wait()
        cv = cv_v[...]

        def grp(g, c2):
            erows = g * LANES + _iota16()
            valid = (base + erows) < ETOT
            scol = (src_v[pl.ds(g * LANES, LANES)] & 15) * 8
            cnt = plsc.load_gather(stb, [erows, scol + 4])
            al = []
            for h in range(HEADS):
                ex = jnp.exp(sb[h, pl.ds(g * LANES, LANES)] - cv)
                den = plsc.load_gather(stb, [erows, scol + h])
                a = ex / (den + EPS) * cnt
                al.append(jnp.where(valid, a, 0.0))
            for dd in range(HID):
                col = jnp.full((LANES,), dd, jnp.int32)
                o = (plsc.load_gather(mb, [erows, col])
                     + plsc.load_gather(meb, [erows, col])) * al[dd // DPH]
                plsc.store_scatter(ob, [erows, col], o)
            return c2

        lax.fori_loop(0, CHC // LANES, grp, 0)
        pltpu.sync_copy(ob, aggr_sp.at[dst_v], add=True)
        return carry

    lax.fori_loop(0, NCHUNKC, chunk, 0)
    plsc.subcore_barrier()

    @pl.when(sid == 0)
    def _():
        pltpu.sync_copy(aggr_sp, aggr2_hbm.at[cid])


_apply_call = pl.kernel(
    _apply_body,
    out_type=jax.ShapeDtypeStruct((NC, NNODE, HID), jnp.float32),
    mesh=_sc_mesh,
    compiler_params=pltpu.CompilerParams(needs_layout_passes=False),
    scratch_types=[
        pltpu.VMEM((CHC,), jnp.int32),
        pltpu.VMEM((CHC,), jnp.int32),
        pltpu.VMEM((CHC,), jnp.int32),
        pltpu.VMEM((CHC,), jnp.int32),
        pltpu.VMEM((HEADS, CHC), jnp.float32),
        pltpu.VMEM((CHC, HID), jnp.float32),
        pltpu.VMEM((CHC, HID), jnp.float32),
        pltpu.VMEM((CHC, HID), jnp.float32),
        pltpu.VMEM((CHC, HID), jnp.float32),
        pltpu.VMEM((LANES,), jnp.float32),
        pltpu.VMEM_SHARED((NNODE, HID), jnp.float32),
        pltpu.SemaphoreType.DMA,
    ],
)


def _edge_class_inputs():
    ets = np.repeat(np.arange(NETY + 1), NTY * NTY)
    nth = np.tile(np.repeat(np.arange(NTY), NTY), NETY + 1)
    ntt = np.tile(np.arange(NTY), (NETY + 1) * NTY)
    ee = np.zeros((NCLS, NETY + 1 + 2 * NTY), np.float32)
    ee[np.arange(NCLS), ets] = 1.0
    ee[np.arange(NCLS), NETY + 1 + nth] = 1.0
    ee[np.arange(NCLS), NETY + 1 + NTY + ntt] = 1.0
    return jnp.asarray(ee)


_USE_SC = frozenset({"cls", "score", "denom", "apply"})


def kernel(H, edge_index, edge_type, node_type, node_score, Wnt, bnt, Wsc, bsc,
           We1, be1, We2, be2, Wk, bk, Wm, bm, Wq, bq, W1, b1, W2, b2,
           VhW, Vhb, VxW, Vxb):
    Bs, n_node, d = H.shape
    ntf = node_type.reshape(-1).astype(jnp.int32)
    js = jnp.power(1.1, jnp.arange(HID // 2, dtype=jnp.float32))[None, :]
    eein = _edge_class_inputs()

    nfe, emb = _prep(ntf[:, None], node_score.reshape(-1, 1), js, eein,
                     Wnt, bnt[None], Wsc, bsc[None], We1, be1[None], We2, be2[None])

    pad = EPAD - ETOT
    loop = jnp.arange(NNODE, dtype=jnp.int32)
    zpad = jnp.zeros((pad,), jnp.int32)
    srcf = jnp.concatenate([edge_index[0].astype(jnp.int32), loop, zpad])
    dstf = jnp.concatenate([edge_index[1].astype(jnp.int32), loop, zpad])
    etf = jnp.concatenate([edge_type.astype(jnp.int32),
                           jnp.full((NNODE,), NETY, jnp.int32), zpad])
    zn128 = jnp.zeros((NNODE, HID), jnp.float32)

    validm = jnp.arange(EPAD) < ETOT
    if "cls" in _USE_SC:
        cls, cntp = _cls_call(srcf, dstf, etf, ntf)
        cnt = cntp.reshape(NW, NNODE).sum(axis=0)
    else:
        cls = etf * (NTY * NTY) + ntf[srcf] * NTY + ntf[dstf]
        cnt = jax.ops.segment_sum(jnp.where(validm, 1.0, 0.0), srcf, num_segments=NNODE)

    x = H.reshape(-1, d)
    for i in range(2):
        qn, kn, mn, ke, me = _tables(
            x, nfe, Wq[i], bq[i][None], Wk[i][:2 * HID], bk[i][None],
            Wk[i][2 * HID:], Wm[i][:2 * HID], bm[i][None], Wm[i][2 * HID:], emb)
        if "score" in _USE_SC:
            scores, smaxp = _score_call(srcf, dstf, cls, qn, kn, ke)
            cvec = jnp.full((LANES,), jnp.max(smaxp), jnp.float32)
        else:
            q = qn[srcf].reshape(-1, HEADS, DPH)
            kv = (kn[dstf] + ke[cls]).reshape(-1, HEADS, DPH)
            scores = jnp.sum(q * kv, axis=2).T
            cvec = jnp.full((LANES,), jnp.max(scores), jnp.float32)
        if "denom" in _USE_SC:
            denp = _denom_call(scores, srcf, cvec)
            den = denp.reshape(NW, NNODE, HEADS).sum(axis=0)
        else:
            exx = jnp.where(validm[:, None], jnp.exp(scores.T - cvec[0]), 0.0)
            den = jax.ops.segment_sum(exx, srcf, num_segments=NNODE)
        stats = jnp.concatenate(
            [den, cnt[:, None],
             jnp.zeros((NNODE, 3), jnp.float32)], axis=1).reshape(NNODE // LANES, LANES * 8)
        if "apply" in _USE_SC:
            aggr2 = _apply_call(srcf, dstf, cls, scores, cvec, stats, mn, me, zn128)
        else:
            exx = jnp.where(validm[:, None], jnp.exp(scores.T - cvec[0]), 0.0)
            alpha = exx / (den[srcf] + EPS) * cnt[srcf][:, None]
            msg = (mn[srcf] + me[cls]).reshape(-1, HEADS, DPH)
            outе = (msg * alpha[:, :, None]).reshape(-1, HID)
            aggr = jax.ops.segment_sum(outе, dstf, num_segments=NNODE)
            aggr2 = jnp.stack([aggr, jnp.zeros_like(aggr)])
        x = _post(aggr2, W1[i], b1[i][None], W2[i], b2[i][None])

    out = _final(H.reshape(-1, d), x, VhW, Vhb[None], VxW, Vxb[None])
    return out.reshape(Bs, n_node, d)
